# Initial kernel scaffold; baseline (speedup 1.0000x reference)
import jax
import jax.numpy as jnp
from jax.experimental import pallas as pl


def kernel(node_features, edge_index, edge_type_or_attr, batch_index, W0, b0, ln_w, ln_b, Ws, bs, t):
    raise NotImplementedError("write your pallas kernel here")


# speedup vs baseline: 8.5892x; 8.5892x over previous
import functools

import jax
import jax.numpy as jnp
from jax import lax
from jax.experimental import pallas as pl
from jax.experimental.pallas import tpu as pltpu
from jax.experimental.pallas import tpu_sc as plsc

NC, NS = 2, 16
NW = NC * NS
EB = 128
RB = 2000

_HI = lax.Precision.HIGHEST


def _dot(a, b, dims=None):
    if dims is None:
        return jnp.dot(a, b, preferred_element_type=jnp.float32, precision=_HI)
    return lax.dot_general(a, b, (dims, ((), ())),
                           preferred_element_type=jnp.float32, precision=_HI)


@functools.lru_cache(maxsize=None)
def _make_scatter_kernel(n_pad, e_pad, d):
    epw = e_pad // NW
    nb = epw // EB
    rps = n_pad // NS

    mesh = plsc.VectorSubcoreMesh(core_axis_name="c", subcore_axis_name="s",
                                  num_cores=NC, num_subcores=NS)

    @functools.partial(
        pl.kernel,
        mesh=mesh,
        compiler_params=pltpu.CompilerParams(use_tc_tiling_on_sc=False),
        out_type=jax.ShapeDtypeStruct((NC, n_pad, d), jnp.float32),
        scratch_types=[
            pltpu.VMEM((EB,), jnp.int32),
            pltpu.VMEM((EB,), jnp.int32),
            pltpu.VMEM((EB, d), jnp.float32),
            pltpu.VMEM_SHARED((n_pad, d), jnp.float32),
            pltpu.SemaphoreType.DMA,
            pltpu.SemaphoreType.DMA,
        ],
    )
    def k(src_hbm, dst_hbm, table_hbm, zeros_hbm, out_hbm,
          sidx, didx, rows, acc, gsem, zsem):
        c = lax.axis_index("c")
        s = lax.axis_index("s")
        wid = c * NS + s
        r0 = s * rps
        pltpu.async_copy(zeros_hbm.at[pl.ds(r0, rps)], acc.at[pl.ds(r0, rps)],
                         zsem).wait()
        plsc.subcore_barrier()
        base = wid * epw

        def body(i, carry):
            off = base + i * EB
            pltpu.async_copy(src_hbm.at[pl.ds(off, EB)], sidx, gsem).wait()
            pltpu.async_copy(dst_hbm.at[pl.ds(off, EB)], didx, gsem).wait()
            pltpu.async_copy(table_hbm.at[sidx], rows, gsem).wait()
            pltpu.sync_copy(rows, acc.at[didx], add=True)
            return carry

        lax.fori_loop(0, nb, body, 0)
        plsc.subcore_barrier()
        pltpu.async_copy(acc.at[pl.ds(r0, rps)], out_hbm.at[c, pl.ds(r0, rps)],
                         zsem).wait()

    return k


def _tc_first(x, w0, d0, d1):
    n, din = x.shape
    dh = w0.shape[1]
    grid = (n // RB,)

    def body(x_ref, w_ref, d0_ref, d1_ref, g_ref, dis_ref):
        deg = 1.0 + d0_ref[:, 0:1] + d1_ref[:, 0:1]
        dis = lax.rsqrt(deg)
        h = _dot(x_ref[...], w_ref[...])
        g_ref[...] = h * dis
        dis_ref[...] = jnp.broadcast_to(dis, (RB, dh))

    return pl.pallas_call(
        body,
        grid=grid,
        in_specs=[
            pl.BlockSpec((RB, din), lambda i: (i, 0)),
            pl.BlockSpec((din, dh), lambda i: (0, 0)),
            pl.BlockSpec((RB, 8), lambda i: (i, 0)),
            pl.BlockSpec((RB, 8), lambda i: (i, 0)),
        ],
        out_specs=[
            pl.BlockSpec((RB, dh), lambda i: (i, 0)),
            pl.BlockSpec((RB, dh), lambda i: (i, 0)),
        ],
        out_shape=[
            jax.ShapeDtypeStruct((n, dh), jnp.float32),
            jax.ShapeDtypeStruct((n, dh), jnp.float32),
        ],
    )(x, w0, d0, d1)


def _tc_inter(a0, a1, g_prev, dis64, beta, lnw, lnb, w):
    n, dh = g_prev.shape

    def body(a0_ref, a1_ref, g_ref, dis_ref, beta_ref, lnw_ref, lnb_ref, w_ref,
             r_ref, gn_ref):
        dis = dis_ref[...]
        r = dis * (a0_ref[...] + a1_ref[...] + g_ref[...]) + beta_ref[...]
        r_ref[...] = r
        mu = jnp.mean(r, axis=-1, keepdims=True)
        cen = r - mu
        var = jnp.mean(cen * cen, axis=-1, keepdims=True)
        hn = cen * lax.rsqrt(var + 1e-5) * lnw_ref[...] + lnb_ref[...]
        h = jnp.where(hn >= 0, hn, 0.01 * hn)
        gn_ref[...] = dis * _dot(h, w_ref[...])

    return pl.pallas_call(
        body,
        grid=(n // RB,),
        in_specs=[
            pl.BlockSpec((RB, dh), lambda i: (i, 0)),
            pl.BlockSpec((RB, dh), lambda i: (i, 0)),
            pl.BlockSpec((RB, dh), lambda i: (i, 0)),
            pl.BlockSpec((RB, dh), lambda i: (i, 0)),
            pl.BlockSpec((1, dh), lambda i: (0, 0)),
            pl.BlockSpec((1, dh), lambda i: (0, 0)),
            pl.BlockSpec((1, dh), lambda i: (0, 0)),
            pl.BlockSpec((dh, dh), lambda i: (0, 0)),
        ],
        out_specs=[
            pl.BlockSpec((RB, dh), lambda i: (i, 0)),
            pl.BlockSpec((RB, dh), lambda i: (i, 0)),
        ],
        out_shape=[
            jax.ShapeDtypeStruct((n, dh), jnp.float32),
            jax.ShapeDtypeStruct((n, dh), jnp.float32),
        ],
    )(a0, a1, g_prev, dis64, beta, lnw, lnb, w)


def _tc_final(a0, a1, g_prev, dis64, beta):
    n, dh = g_prev.shape

    def body(a0_ref, a1_ref, g_ref, dis_ref, beta_ref, r_ref):
        r_ref[...] = dis_ref[...] * (a0_ref[...] + a1_ref[...] + g_ref[...]) \
            + beta_ref[...]

    return pl.pallas_call(
        body,
        grid=(n // RB,),
        in_specs=[
            pl.BlockSpec((RB, dh), lambda i: (i, 0)),
            pl.BlockSpec((RB, dh), lambda i: (i, 0)),
            pl.BlockSpec((RB, dh), lambda i: (i, 0)),
            pl.BlockSpec((RB, dh), lambda i: (i, 0)),
            pl.BlockSpec((1, dh), lambda i: (0, 0)),
        ],
        out_specs=[pl.BlockSpec((RB, dh), lambda i: (i, 0))],
        out_shape=[jax.ShapeDtypeStruct((n, dh), jnp.float32)],
    )(a0, a1, g_prev, dis64, beta)[0]


def _softmax_stats(x, batch_col, t, g):
    n, dtot = x.shape

    def body(x_ref, b_ref, t_ref, s1_ref, cnt_ref):
        i = pl.program_id(0)
        oh = (b_ref[...] == lax.broadcasted_iota(jnp.int32, (1, g), 1))
        oh = oh.astype(jnp.float32)
        s = t_ref[0, 0] * x_ref[...]
        p = _dot(oh, s, dims=((0,), (0,)))
        c = jnp.sum(oh, axis=0, keepdims=True)

        @pl.when(i == 0)
        def _():
            s1_ref[...] = p
            cnt_ref[...] = c

        @pl.when(i > 0)
        def _():
            s1_ref[...] += p
            cnt_ref[...] += c

    return pl.pallas_call(
        body,
        grid=(n // RB,),
        in_specs=[
            pl.BlockSpec((RB, dtot), lambda i: (i, 0)),
            pl.BlockSpec((RB, 1), lambda i: (i, 0)),
            pl.BlockSpec((1, 1), lambda i: (0, 0)),
        ],
        out_specs=[
            pl.BlockSpec((g, dtot), lambda i: (0, 0)),
            pl.BlockSpec((1, g), lambda i: (0, 0)),
        ],
        out_shape=[
            jax.ShapeDtypeStruct((g, dtot), jnp.float32),
            jax.ShapeDtypeStruct((1, g), jnp.float32),
        ],
    )(x, batch_col, t)


def _softmax_final(x, batch_col, t, s1, cnt_t, g):
    n, dtot = x.shape
    nblk = n // RB

    def body(x_ref, b_ref, t_ref, s1_ref, cnt_ref, out_ref, num_s, den_s):
        i = pl.program_id(0)
        oh = (b_ref[...] == lax.broadcasted_iota(jnp.int32, (1, g), 1))
        oh = oh.astype(jnp.float32)
        shift = s1_ref[...] / jnp.maximum(cnt_ref[...], 1.0)
        p = _dot(oh, shift)
        xv = x_ref[...]
        e = jnp.exp(t_ref[0, 0] * xv - p)
        num = _dot(oh, e * xv, dims=((0,), (0,)))
        den = _dot(oh, e, dims=((0,), (0,)))

        @pl.when(i == 0)
        def _():
            num_s[...] = num
            den_s[...] = den

        @pl.when(i > 0)
        def _():
            num_s[...] += num
            den_s[...] += den

        @pl.when(i == nblk - 1)
        def _():
            d = den_s[...]
            out_ref[...] = jnp.where(d > 0, num_s[...] / d, 0.0)

    return pl.pallas_call(
        body,
        grid=(nblk,),
        in_specs=[
            pl.BlockSpec((RB, dtot), lambda i: (i, 0)),
            pl.BlockSpec((RB, 1), lambda i: (i, 0)),
            pl.BlockSpec((1, 1), lambda i: (0, 0)),
            pl.BlockSpec((g, dtot), lambda i: (0, 0)),
            pl.BlockSpec((g, 1), lambda i: (0, 0)),
        ],
        out_specs=[pl.BlockSpec((g, dtot), lambda i: (0, 0))],
        out_shape=[jax.ShapeDtypeStruct((g, dtot), jnp.float32)],
        scratch_shapes=[
            pltpu.VMEM((g, dtot), jnp.float32),
            pltpu.VMEM((g, dtot), jnp.float32),
        ],
    )(x, batch_col, t, s1, cnt_t)[0]


def kernel(node_features, edge_index, edge_type_or_attr, batch_index,
           W0, b0, ln_w, ln_b, Ws, bs, t):
    n, din = node_features.shape
    e = edge_index.shape[1]
    dh = W0.shape[1]
    nlayers = Ws.shape[0]
    g = 64

    n_pad = ((n + NS * 8) // (NS * 8)) * (NS * 8)
    epw = ((e + NW - 1) // NW + EB - 1) // EB * EB
    e_pad = epw * NW

    src = edge_index[0]
    dst = edge_index[1]
    pad = e_pad - e
    src_p = jnp.concatenate([src, jnp.zeros((pad,), jnp.int32)])
    dst_p = jnp.concatenate([dst, jnp.full((pad,), n_pad - 1, jnp.int32)])

    zeros16 = jnp.zeros((n_pad, 16), jnp.float32)
    zeros64 = jnp.zeros((n_pad, dh), jnp.float32)
    ones_tab = jnp.ones((n, 16), jnp.float32)

    scat16 = _make_scatter_kernel(n_pad, e_pad, 16)
    scat64 = _make_scatter_kernel(n_pad, e_pad, dh)

    deg_parts = scat16(src_p, dst_p, ones_tab, zeros16)
    d0 = deg_parts[0, :n, 0:8]
    d1 = deg_parts[1, :n, 0:8]

    g_cur, dis64 = _tc_first(node_features, W0, d0, d1)

    betas = [b0.reshape(1, dh)] + [bs[i].reshape(1, dh) for i in range(nlayers)]
    results = []
    for k in range(nlayers + 1):
        acc = scat64(src_p, dst_p, g_cur, zeros64)
        a0 = acc[0, :n, :]
        a1 = acc[1, :n, :]
        if k < nlayers:
            r, g_next = _tc_inter(a0, a1, g_cur, dis64, betas[k],
                                  ln_w[k].reshape(1, dh), ln_b[k].reshape(1, dh),
                                  Ws[k])
            results.append(r)
            g_cur = g_next
        else:
            results.append(_tc_final(a0, a1, g_cur, dis64, betas[k]))

    node_repr = jnp.concatenate(results, axis=-1)

    batch_col = batch_index.reshape(n, 1)
    t2 = t.reshape(1, 1)
    s1, cnt = _softmax_stats(node_repr, batch_col, t2, g)
    graph_repr = _softmax_final(node_repr, batch_col, t2, s1,
                                cnt.reshape(g, 1), g)
    return (graph_repr, node_repr)


# speedup vs baseline: 8.7375x; 1.0173x over previous
import functools

import jax
import jax.numpy as jnp
from jax import lax
from jax.experimental import pallas as pl
from jax.experimental.pallas import tpu as pltpu
from jax.experimental.pallas import tpu_sc as plsc

NC, NS = 2, 16
NW = NC * NS
EB = 128
RB = 2000

_HI = lax.Precision.HIGHEST


def _dot(a, b, dims=None):
    if dims is None:
        return jnp.dot(a, b, preferred_element_type=jnp.float32, precision=_HI)
    return lax.dot_general(a, b, (dims, ((), ())),
                           preferred_element_type=jnp.float32, precision=_HI)


@functools.lru_cache(maxsize=None)
def _make_scatter_kernel(n_pad, nb, d):
    rps = n_pad // NS

    mesh = plsc.VectorSubcoreMesh(core_axis_name="c", subcore_axis_name="s",
                                  num_cores=NC, num_subcores=NS)

    @functools.partial(
        pl.kernel,
        mesh=mesh,
        compiler_params=pltpu.CompilerParams(use_tc_tiling_on_sc=False),
        out_type=jax.ShapeDtypeStruct((NC, n_pad, d), jnp.float32),
        scratch_types=[
            pltpu.VMEM((nb, EB), jnp.int32),
            pltpu.VMEM((nb, EB), jnp.int32),
            pltpu.VMEM((EB, d), jnp.float32),
            pltpu.VMEM((EB, d), jnp.float32),
            pltpu.VMEM_SHARED((n_pad, d), jnp.float32),
            pltpu.SemaphoreType.DMA,
            pltpu.SemaphoreType.DMA,
            pltpu.SemaphoreType.DMA,
        ],
    )
    def k(src_hbm, dst_hbm, table_hbm, zeros_hbm, out_hbm,
          sidx, didx, rows0, rows1, acc, gsem0, gsem1, zsem):
        c = lax.axis_index("c")
        s = lax.axis_index("s")
        wid = c * NS + s
        r0 = s * rps
        zd = pltpu.async_copy(zeros_hbm.at[pl.ds(r0, rps)],
                              acc.at[pl.ds(r0, rps)], zsem)
        pltpu.async_copy(src_hbm.at[wid], sidx, gsem0).wait()
        pltpu.async_copy(dst_hbm.at[wid], didx, gsem1).wait()
        zd.wait()
        plsc.subcore_barrier()

        pltpu.async_copy(table_hbm.at[sidx.at[0]], rows0, gsem0)

        def body(p, carry):
            j = 2 * p
            pltpu.async_copy(table_hbm.at[sidx.at[j + 1]], rows1, gsem1)
            pltpu.make_async_copy(table_hbm.at[sidx.at[0]], rows0, gsem0).wait()
            pltpu.sync_copy(rows0, acc.at[didx.at[j]], add=True)

            @pl.when(j + 2 < nb)
            def _():
                pltpu.async_copy(table_hbm.at[sidx.at[j + 2]], rows0, gsem0)

            pltpu.make_async_copy(table_hbm.at[sidx.at[0]], rows1, gsem1).wait()
            pltpu.sync_copy(rows1, acc.at[didx.at[j + 1]], add=True)
            return carry

        lax.fori_loop(0, nb // 2, body, 0)
        plsc.subcore_barrier()
        pltpu.async_copy(acc.at[pl.ds(r0, rps)], out_hbm.at[c, pl.ds(r0, rps)],
                         zsem).wait()

    return k


def _tc_first(x, w0, d0, d1):
    n, din = x.shape
    dh = w0.shape[1]
    grid = (n // RB,)

    def body(x_ref, w_ref, d0_ref, d1_ref, g_ref, dis_ref):
        deg = 1.0 + d0_ref[:, 0:1] + d1_ref[:, 0:1]
        dis = lax.rsqrt(deg)
        h = _dot(x_ref[...], w_ref[...])
        g_ref[...] = h * dis
        dis_ref[...] = jnp.broadcast_to(dis, (RB, dh))

    return pl.pallas_call(
        body,
        grid=grid,
        in_specs=[
            pl.BlockSpec((RB, din), lambda i: (i, 0)),
            pl.BlockSpec((din, dh), lambda i: (0, 0)),
            pl.BlockSpec((RB, 8), lambda i: (i, 0)),
            pl.BlockSpec((RB, 8), lambda i: (i, 0)),
        ],
        out_specs=[
            pl.BlockSpec((RB, dh), lambda i: (i, 0)),
            pl.BlockSpec((RB, dh), lambda i: (i, 0)),
        ],
        out_shape=[
            jax.ShapeDtypeStruct((n, dh), jnp.float32),
            jax.ShapeDtypeStruct((n, dh), jnp.float32),
        ],
    )(x, w0, d0, d1)


def _tc_inter(a0, a1, g_prev, dis64, beta, lnw, lnb, w):
    n, dh = g_prev.shape

    def body(a0_ref, a1_ref, g_ref, dis_ref, beta_ref, lnw_ref, lnb_ref, w_ref,
             r_ref, gn_ref):
        dis = dis_ref[...]
        r = dis * (a0_ref[...] + a1_ref[...] + g_ref[...]) + beta_ref[...]
        r_ref[...] = r
        mu = jnp.mean(r, axis=-1, keepdims=True)
        cen = r - mu
        var = jnp.mean(cen * cen, axis=-1, keepdims=True)
        hn = cen * lax.rsqrt(var + 1e-5) * lnw_ref[...] + lnb_ref[...]
        h = jnp.where(hn >= 0, hn, 0.01 * hn)
        gn_ref[...] = dis * _dot(h, w_ref[...])

    return pl.pallas_call(
        body,
        grid=(n // RB,),
        in_specs=[
            pl.BlockSpec((RB, dh), lambda i: (i, 0)),
            pl.BlockSpec((RB, dh), lambda i: (i, 0)),
            pl.BlockSpec((RB, dh), lambda i: (i, 0)),
            pl.BlockSpec((RB, dh), lambda i: (i, 0)),
            pl.BlockSpec((1, dh), lambda i: (0, 0)),
            pl.BlockSpec((1, dh), lambda i: (0, 0)),
            pl.BlockSpec((1, dh), lambda i: (0, 0)),
            pl.BlockSpec((dh, dh), lambda i: (0, 0)),
        ],
        out_specs=[
            pl.BlockSpec((RB, dh), lambda i: (i, 0)),
            pl.BlockSpec((RB, dh), lambda i: (i, 0)),
        ],
        out_shape=[
            jax.ShapeDtypeStruct((n, dh), jnp.float32),
            jax.ShapeDtypeStruct((n, dh), jnp.float32),
        ],
    )(a0, a1, g_prev, dis64, beta, lnw, lnb, w)


def _tc_final(a0, a1, g_prev, dis64, beta):
    n, dh = g_prev.shape

    def body(a0_ref, a1_ref, g_ref, dis_ref, beta_ref, r_ref):
        r_ref[...] = dis_ref[...] * (a0_ref[...] + a1_ref[...] + g_ref[...]) \
            + beta_ref[...]

    return pl.pallas_call(
        body,
        grid=(n // RB,),
        in_specs=[
            pl.BlockSpec((RB, dh), lambda i: (i, 0)),
            pl.BlockSpec((RB, dh), lambda i: (i, 0)),
            pl.BlockSpec((RB, dh), lambda i: (i, 0)),
            pl.BlockSpec((RB, dh), lambda i: (i, 0)),
            pl.BlockSpec((1, dh), lambda i: (0, 0)),
        ],
        out_specs=[pl.BlockSpec((RB, dh), lambda i: (i, 0))],
        out_shape=[jax.ShapeDtypeStruct((n, dh), jnp.float32)],
    )(a0, a1, g_prev, dis64, beta)[0]


def _softmax_stats(x, batch_col, t, g):
    n, dtot = x.shape

    def body(x_ref, b_ref, t_ref, s1_ref, cnt_ref):
        i = pl.program_id(0)
        oh = (b_ref[...] == lax.broadcasted_iota(jnp.int32, (1, g), 1))
        oh = oh.astype(jnp.float32)
        s = t_ref[0, 0] * x_ref[...]
        p = _dot(oh, s, dims=((0,), (0,)))
        c = jnp.sum(oh, axis=0, keepdims=True)

        @pl.when(i == 0)
        def _():
            s1_ref[...] = p
            cnt_ref[...] = c

        @pl.when(i > 0)
        def _():
            s1_ref[...] += p
            cnt_ref[...] += c

    return pl.pallas_call(
        body,
        grid=(n // RB,),
        in_specs=[
            pl.BlockSpec((RB, dtot), lambda i: (i, 0)),
            pl.BlockSpec((RB, 1), lambda i: (i, 0)),
            pl.BlockSpec((1, 1), lambda i: (0, 0)),
        ],
        out_specs=[
            pl.BlockSpec((g, dtot), lambda i: (0, 0)),
            pl.BlockSpec((1, g), lambda i: (0, 0)),
        ],
        out_shape=[
            jax.ShapeDtypeStruct((g, dtot), jnp.float32),
            jax.ShapeDtypeStruct((1, g), jnp.float32),
        ],
    )(x, batch_col, t)


def _softmax_final(x, batch_col, t, s1, cnt_t, g):
    n, dtot = x.shape
    nblk = n // RB

    def body(x_ref, b_ref, t_ref, s1_ref, cnt_ref, out_ref, num_s, den_s):
        i = pl.program_id(0)
        oh = (b_ref[...] == lax.broadcasted_iota(jnp.int32, (1, g), 1))
        oh = oh.astype(jnp.float32)
        shift = s1_ref[...] / jnp.maximum(cnt_ref[...], 1.0)
        p = _dot(oh, shift)
        xv = x_ref[...]
        e = jnp.exp(t_ref[0, 0] * xv - p)
        num = _dot(oh, e * xv, dims=((0,), (0,)))
        den = _dot(oh, e, dims=((0,), (0,)))

        @pl.when(i == 0)
        def _():
            num_s[...] = num
            den_s[...] = den

        @pl.when(i > 0)
        def _():
            num_s[...] += num
            den_s[...] += den

        @pl.when(i == nblk - 1)
        def _():
            d = den_s[...]
            out_ref[...] = jnp.where(d > 0, num_s[...] / d, 0.0)

    return pl.pallas_call(
        body,
        grid=(nblk,),
        in_specs=[
            pl.BlockSpec((RB, dtot), lambda i: (i, 0)),
            pl.BlockSpec((RB, 1), lambda i: (i, 0)),
            pl.BlockSpec((1, 1), lambda i: (0, 0)),
            pl.BlockSpec((g, dtot), lambda i: (0, 0)),
            pl.BlockSpec((g, 1), lambda i: (0, 0)),
        ],
        out_specs=[pl.BlockSpec((g, dtot), lambda i: (0, 0))],
        out_shape=[jax.ShapeDtypeStruct((g, dtot), jnp.float32)],
        scratch_shapes=[
            pltpu.VMEM((g, dtot), jnp.float32),
            pltpu.VMEM((g, dtot), jnp.float32),
        ],
    )(x, batch_col, t, s1, cnt_t)[0]


def kernel(node_features, edge_index, edge_type_or_attr, batch_index,
           W0, b0, ln_w, ln_b, Ws, bs, t):
    n, din = node_features.shape
    e = edge_index.shape[1]
    dh = W0.shape[1]
    nlayers = Ws.shape[0]
    g = 64

    n_pad = ((n + NS * 8) // (NS * 8)) * (NS * 8)
    epw = ((e + NW - 1) // NW + 2 * EB - 1) // (2 * EB) * (2 * EB)
    e_pad = epw * NW
    nb = epw // EB

    src = edge_index[0]
    dst = edge_index[1]
    pad = e_pad - e
    src_p = jnp.concatenate([src, jnp.zeros((pad,), jnp.int32)])
    dst_p = jnp.concatenate([dst, jnp.full((pad,), n_pad - 1, jnp.int32)])
    src_p = src_p.reshape(NW, nb, EB)
    dst_p = dst_p.reshape(NW, nb, EB)

    zeros16 = jnp.zeros((n_pad, 16), jnp.float32)
    zeros64 = jnp.zeros((n_pad, dh), jnp.float32)
    ones_tab = jnp.ones((n, 16), jnp.float32)

    scat16 = _make_scatter_kernel(n_pad, nb, 16)
    scat64 = _make_scatter_kernel(n_pad, nb, dh)

    deg_parts = scat16(src_p, dst_p, ones_tab, zeros16)
    d0 = deg_parts[0, :n, 0:8]
    d1 = deg_parts[1, :n, 0:8]

    g_cur, dis64 = _tc_first(node_features, W0, d0, d1)

    betas = [b0.reshape(1, dh)] + [bs[i].reshape(1, dh) for i in range(nlayers)]
    results = []
    for k in range(nlayers + 1):
        acc = scat64(src_p, dst_p, g_cur, zeros64)
        a0 = acc[0, :n, :]
        a1 = acc[1, :n, :]
        if k < nlayers:
            r, g_next = _tc_inter(a0, a1, g_cur, dis64, betas[k],
                                  ln_w[k].reshape(1, dh), ln_b[k].reshape(1, dh),
                                  Ws[k])
            results.append(r)
            g_cur = g_next
        else:
            results.append(_tc_final(a0, a1, g_cur, dis64, betas[k]))

    node_repr = jnp.concatenate(results, axis=-1)

    batch_col = batch_index.reshape(n, 1)
    t2 = t.reshape(1, 1)
    s1, cnt = _softmax_stats(node_repr, batch_col, t2, g)
    graph_repr = _softmax_final(node_repr, batch_col, t2, s1,
                                cnt.reshape(g, 1), g)
    return (graph_repr, node_repr)


# speedup vs baseline: 8.7527x; 1.0017x over previous
import functools

import jax
import jax.numpy as jnp
from jax import lax
from jax.experimental import pallas as pl
from jax.experimental.pallas import tpu as pltpu
from jax.experimental.pallas import tpu_sc as plsc

NC, NS = 2, 16
NW = NC * NS
EB = 128
NBUF = 4
RB = 2000

_HI = lax.Precision.HIGHEST


def _dot(a, b, dims=None):
    if dims is None:
        return jnp.dot(a, b, preferred_element_type=jnp.float32, precision=_HI)
    return lax.dot_general(a, b, (dims, ((), ())),
                           preferred_element_type=jnp.float32, precision=_HI)


@functools.lru_cache(maxsize=None)
def _make_scatter_kernel(n_pad, nb, d):
    rps = n_pad // NS

    mesh = plsc.VectorSubcoreMesh(core_axis_name="c", subcore_axis_name="s",
                                  num_cores=NC, num_subcores=NS)

    @functools.partial(
        pl.kernel,
        mesh=mesh,
        compiler_params=pltpu.CompilerParams(use_tc_tiling_on_sc=False),
        out_type=jax.ShapeDtypeStruct((NC, n_pad, d), jnp.float32),
        scratch_types=[
            pltpu.VMEM((nb, EB), jnp.int32),
            pltpu.VMEM((nb, EB), jnp.int32),
            [pltpu.VMEM((EB, d), jnp.float32) for _ in range(NBUF)],
            pltpu.VMEM_SHARED((n_pad, d), jnp.float32),
            [pltpu.SemaphoreType.DMA for _ in range(NBUF)],
            [pltpu.SemaphoreType.DMA for _ in range(NBUF)],
            pltpu.SemaphoreType.DMA,
        ],
    )
    def k(src_hbm, dst_hbm, table_hbm, zeros_hbm, out_hbm,
          sidx, didx, rows, acc, gsems, ssems, zsem):
        c = lax.axis_index("c")
        s = lax.axis_index("s")
        wid = c * NS + s
        r0 = s * rps
        zd = pltpu.async_copy(zeros_hbm.at[pl.ds(r0, rps)],
                              acc.at[pl.ds(r0, rps)], zsem)
        pltpu.async_copy(src_hbm.at[wid], sidx, gsems[0]).wait()
        pltpu.async_copy(dst_hbm.at[wid], didx, gsems[1]).wait()
        zd.wait()
        plsc.subcore_barrier()

        def gather(j, u):
            pltpu.async_copy(table_hbm.at[sidx.at[j]], rows[u], gsems[u])

        def wait_gather(u):
            pltpu.make_async_copy(table_hbm.at[sidx.at[0]], rows[u],
                                  gsems[u]).wait()

        def scatter(j, u):
            pltpu.async_copy(rows[u], acc.at[didx.at[j]], ssems[u], add=True)

        def wait_scatter(u):
            pltpu.make_async_copy(rows[u], acc.at[didx.at[0]], ssems[u]).wait()

        for u in range(NBUF):
            gather(u, u)

        def body(p, carry):
            j = NBUF * p
            for u in range(NBUF):
                wait_gather(u)
                scatter(j + u, u)
            nxt = j + NBUF

            @pl.when(nxt < nb)
            def _():
                for u in range(NBUF):
                    wait_scatter(u)
                    gather(nxt + u, u)

            return carry

        lax.fori_loop(0, nb // NBUF, body, 0)
        for u in range(NBUF):
            wait_scatter(u)
        plsc.subcore_barrier()
        pltpu.async_copy(acc.at[pl.ds(r0, rps)], out_hbm.at[c, pl.ds(r0, rps)],
                         zsem).wait()

    return k


def _tc_first(x, w0, d0, d1):
    n, din = x.shape
    dh = w0.shape[1]
    grid = (n // RB,)

    def body(x_ref, w_ref, d0_ref, d1_ref, g_ref, dis_ref):
        deg = 1.0 + d0_ref[:, 0:1] + d1_ref[:, 0:1]
        dis = lax.rsqrt(deg)
        h = _dot(x_ref[...], w_ref[...])
        g_ref[...] = h * dis
        dis_ref[...] = jnp.broadcast_to(dis, (RB, dh))

    return pl.pallas_call(
        body,
        grid=grid,
        in_specs=[
            pl.BlockSpec((RB, din), lambda i: (i, 0)),
            pl.BlockSpec((din, dh), lambda i: (0, 0)),
            pl.BlockSpec((RB, 8), lambda i: (i, 0)),
            pl.BlockSpec((RB, 8), lambda i: (i, 0)),
        ],
        out_specs=[
            pl.BlockSpec((RB, dh), lambda i: (i, 0)),
            pl.BlockSpec((RB, dh), lambda i: (i, 0)),
        ],
        out_shape=[
            jax.ShapeDtypeStruct((n, dh), jnp.float32),
            jax.ShapeDtypeStruct((n, dh), jnp.float32),
        ],
    )(x, w0, d0, d1)


def _tc_inter(a0, a1, g_prev, dis64, beta, lnw, lnb, w):
    n, dh = g_prev.shape

    def body(a0_ref, a1_ref, g_ref, dis_ref, beta_ref, lnw_ref, lnb_ref, w_ref,
             r_ref, gn_ref):
        dis = dis_ref[...]
        r = dis * (a0_ref[...] + a1_ref[...] + g_ref[...]) + beta_ref[...]
        r_ref[...] = r
        mu = jnp.mean(r, axis=-1, keepdims=True)
        cen = r - mu
        var = jnp.mean(cen * cen, axis=-1, keepdims=True)
        hn = cen * lax.rsqrt(var + 1e-5) * lnw_ref[...] + lnb_ref[...]
        h = jnp.where(hn >= 0, hn, 0.01 * hn)
        gn_ref[...] = dis * _dot(h, w_ref[...])

    return pl.pallas_call(
        body,
        grid=(n // RB,),
        in_specs=[
            pl.BlockSpec((RB, dh), lambda i: (i, 0)),
            pl.BlockSpec((RB, dh), lambda i: (i, 0)),
            pl.BlockSpec((RB, dh), lambda i: (i, 0)),
            pl.BlockSpec((RB, dh), lambda i: (i, 0)),
            pl.BlockSpec((1, dh), lambda i: (0, 0)),
            pl.BlockSpec((1, dh), lambda i: (0, 0)),
            pl.BlockSpec((1, dh), lambda i: (0, 0)),
            pl.BlockSpec((dh, dh), lambda i: (0, 0)),
        ],
        out_specs=[
            pl.BlockSpec((RB, dh), lambda i: (i, 0)),
            pl.BlockSpec((RB, dh), lambda i: (i, 0)),
        ],
        out_shape=[
            jax.ShapeDtypeStruct((n, dh), jnp.float32),
            jax.ShapeDtypeStruct((n, dh), jnp.float32),
        ],
    )(a0, a1, g_prev, dis64, beta, lnw, lnb, w)


def _tc_final(a0, a1, g_prev, dis64, beta):
    n, dh = g_prev.shape

    def body(a0_ref, a1_ref, g_ref, dis_ref, beta_ref, r_ref):
        r_ref[...] = dis_ref[...] * (a0_ref[...] + a1_ref[...] + g_ref[...]) \
            + beta_ref[...]

    return pl.pallas_call(
        body,
        grid=(n // RB,),
        in_specs=[
            pl.BlockSpec((RB, dh), lambda i: (i, 0)),
            pl.BlockSpec((RB, dh), lambda i: (i, 0)),
            pl.BlockSpec((RB, dh), lambda i: (i, 0)),
            pl.BlockSpec((RB, dh), lambda i: (i, 0)),
            pl.BlockSpec((1, dh), lambda i: (0, 0)),
        ],
        out_specs=[pl.BlockSpec((RB, dh), lambda i: (i, 0))],
        out_shape=[jax.ShapeDtypeStruct((n, dh), jnp.float32)],
    )(a0, a1, g_prev, dis64, beta)[0]


def _softmax_stats(x, batch_col, t, g):
    n, dtot = x.shape

    def body(x_ref, b_ref, t_ref, s1_ref, cnt_ref):
        i = pl.program_id(0)
        oh = (b_ref[...] == lax.broadcasted_iota(jnp.int32, (1, g), 1))
        oh = oh.astype(jnp.float32)
        s = t_ref[0, 0] * x_ref[...]
        p = _dot(oh, s, dims=((0,), (0,)))
        c = jnp.sum(oh, axis=0, keepdims=True)

        @pl.when(i == 0)
        def _():
            s1_ref[...] = p
            cnt_ref[...] = c

        @pl.when(i > 0)
        def _():
            s1_ref[...] += p
            cnt_ref[...] += c

    return pl.pallas_call(
        body,
        grid=(n // RB,),
        in_specs=[
            pl.BlockSpec((RB, dtot), lambda i: (i, 0)),
            pl.BlockSpec((RB, 1), lambda i: (i, 0)),
            pl.BlockSpec((1, 1), lambda i: (0, 0)),
        ],
        out_specs=[
            pl.BlockSpec((g, dtot), lambda i: (0, 0)),
            pl.BlockSpec((1, g), lambda i: (0, 0)),
        ],
        out_shape=[
            jax.ShapeDtypeStruct((g, dtot), jnp.float32),
            jax.ShapeDtypeStruct((1, g), jnp.float32),
        ],
    )(x, batch_col, t)


def _softmax_final(x, batch_col, t, s1, cnt_t, g):
    n, dtot = x.shape
    nblk = n // RB

    def body(x_ref, b_ref, t_ref, s1_ref, cnt_ref, out_ref, num_s, den_s):
        i = pl.program_id(0)
        oh = (b_ref[...] == lax.broadcasted_iota(jnp.int32, (1, g), 1))
        oh = oh.astype(jnp.float32)
        shift = s1_ref[...] / jnp.maximum(cnt_ref[...], 1.0)
        p = _dot(oh, shift)
        xv = x_ref[...]
        e = jnp.exp(t_ref[0, 0] * xv - p)
        num = _dot(oh, e * xv, dims=((0,), (0,)))
        den = _dot(oh, e, dims=((0,), (0,)))

        @pl.when(i == 0)
        def _():
            num_s[...] = num
            den_s[...] = den

        @pl.when(i > 0)
        def _():
            num_s[...] += num
            den_s[...] += den

        @pl.when(i == nblk - 1)
        def _():
            d = den_s[...]
            out_ref[...] = jnp.where(d > 0, num_s[...] / d, 0.0)

    return pl.pallas_call(
        body,
        grid=(nblk,),
        in_specs=[
            pl.BlockSpec((RB, dtot), lambda i: (i, 0)),
            pl.BlockSpec((RB, 1), lambda i: (i, 0)),
            pl.BlockSpec((1, 1), lambda i: (0, 0)),
            pl.BlockSpec((g, dtot), lambda i: (0, 0)),
            pl.BlockSpec((g, 1), lambda i: (0, 0)),
        ],
        out_specs=[pl.BlockSpec((g, dtot), lambda i: (0, 0))],
        out_shape=[jax.ShapeDtypeStruct((g, dtot), jnp.float32)],
        scratch_shapes=[
            pltpu.VMEM((g, dtot), jnp.float32),
            pltpu.VMEM((g, dtot), jnp.float32),
        ],
    )(x, batch_col, t, s1, cnt_t)[0]


def kernel(node_features, edge_index, edge_type_or_attr, batch_index,
           W0, b0, ln_w, ln_b, Ws, bs, t):
    n, din = node_features.shape
    e = edge_index.shape[1]
    dh = W0.shape[1]
    nlayers = Ws.shape[0]
    g = 64

    n_pad = ((n + NS * 8) // (NS * 8)) * (NS * 8)
    epw = ((e + NW - 1) // NW + NBUF * EB - 1) // (NBUF * EB) * (NBUF * EB)
    e_pad = epw * NW
    nb = epw // EB

    src = edge_index[0]
    dst = edge_index[1]
    pad = e_pad - e
    src_p = jnp.concatenate([src, jnp.zeros((pad,), jnp.int32)])
    dst_p = jnp.concatenate([dst, jnp.full((pad,), n_pad - 1, jnp.int32)])
    src_p = src_p.reshape(NW, nb, EB)
    dst_p = dst_p.reshape(NW, nb, EB)

    zeros16 = jnp.zeros((n_pad, 16), jnp.float32)
    zeros64 = jnp.zeros((n_pad, dh), jnp.float32)
    ones_tab = jnp.ones((n, 16), jnp.float32)

    scat16 = _make_scatter_kernel(n_pad, nb, 16)
    scat64 = _make_scatter_kernel(n_pad, nb, dh)

    deg_parts = scat16(src_p, dst_p, ones_tab, zeros16)
    d0 = deg_parts[0, :n, 0:8]
    d1 = deg_parts[1, :n, 0:8]

    g_cur, dis64 = _tc_first(node_features, W0, d0, d1)

    betas = [b0.reshape(1, dh)] + [bs[i].reshape(1, dh) for i in range(nlayers)]
    results = []
    for k in range(nlayers + 1):
        acc = scat64(src_p, dst_p, g_cur, zeros64)
        a0 = acc[0, :n, :]
        a1 = acc[1, :n, :]
        if k < nlayers:
            r, g_next = _tc_inter(a0, a1, g_cur, dis64, betas[k],
                                  ln_w[k].reshape(1, dh), ln_b[k].reshape(1, dh),
                                  Ws[k])
            results.append(r)
            g_cur = g_next
        else:
            results.append(_tc_final(a0, a1, g_cur, dis64, betas[k]))

    node_repr = jnp.concatenate(results, axis=-1)

    batch_col = batch_index.reshape(n, 1)
    t2 = t.reshape(1, 1)
    s1, cnt = _softmax_stats(node_repr, batch_col, t2, g)
    graph_repr = _softmax_final(node_repr, batch_col, t2, s1,
                                cnt.reshape(g, 1), g)
    return (graph_repr, node_repr)


# speedup vs baseline: 19.2027x; 2.1939x over previous
import functools

import jax
import jax.numpy as jnp
from jax import lax
from jax.experimental import pallas as pl
from jax.experimental.pallas import tpu as pltpu
from jax.experimental.pallas import tpu_sc as plsc

NC, NS = 2, 16
NW = NC * NS
EB = 128
NBUF = 2
RB = 2000

_HI = lax.Precision.HIGHEST


def _dot(a, b, dims=None):
    if dims is None:
        return jnp.dot(a, b, preferred_element_type=jnp.float32, precision=_HI)
    return lax.dot_general(a, b, (dims, ((), ())),
                           preferred_element_type=jnp.float32, precision=_HI)


@functools.lru_cache(maxsize=None)
def _make_deg_kernel(n_pad, nb):
    rps = n_pad // NS
    mesh = plsc.VectorSubcoreMesh(core_axis_name="c", subcore_axis_name="s",
                                  num_cores=NC, num_subcores=NS)

    @functools.partial(
        pl.kernel,
        mesh=mesh,
        compiler_params=pltpu.CompilerParams(use_tc_tiling_on_sc=False),
        out_type=jax.ShapeDtypeStruct((NC, n_pad, 16), jnp.float32),
        scratch_types=[
            pltpu.VMEM((nb, EB), jnp.int32),
            pltpu.VMEM((EB, 16), jnp.float32),
            pltpu.VMEM_SHARED((n_pad, 16), jnp.float32),
            [pltpu.SemaphoreType.DMA for _ in range(NBUF)],
            pltpu.SemaphoreType.DMA,
        ],
    )
    def k(dst_hbm, ones_hbm, zeros_hbm, out_hbm, didx, ones_v, acc, ssems,
          zsem):
        c = lax.axis_index("c")
        s = lax.axis_index("s")
        wid = c * NS + s
        r0 = s * rps
        zd = pltpu.async_copy(zeros_hbm.at[pl.ds(r0, rps)],
                              acc.at[pl.ds(r0, rps)], zsem)
        pltpu.async_copy(ones_hbm, ones_v, ssems[0]).wait()
        pltpu.async_copy(dst_hbm.at[wid], didx, ssems[0]).wait()
        zd.wait()
        plsc.subcore_barrier()

        def wait_scatter(u):
            pltpu.make_async_copy(ones_v, acc.at[didx.at[0]], ssems[u]).wait()

        def body(p, carry):
            j = NBUF * p
            for u in range(NBUF):
                pltpu.async_copy(ones_v, acc.at[didx.at[j + u]], ssems[u],
                                 add=True)
            for u in range(NBUF):
                wait_scatter(u)
            return carry

        lax.fori_loop(0, nb // NBUF, body, 0)
        plsc.subcore_barrier()
        pltpu.async_copy(acc.at[pl.ds(r0, rps)], out_hbm.at[c, pl.ds(r0, rps)],
                         zsem).wait()

    return k


@functools.lru_cache(maxsize=None)
def _make_scatter_kernel(n_pad, nb, d, n_tab):
    rps = n_pad // NS

    mesh = plsc.VectorSubcoreMesh(core_axis_name="c", subcore_axis_name="s",
                                  num_cores=NC, num_subcores=NS)

    @functools.partial(
        pl.kernel,
        mesh=mesh,
        compiler_params=pltpu.CompilerParams(use_tc_tiling_on_sc=False),
        out_type=jax.ShapeDtypeStruct((NC, n_pad, d), jnp.float32),
        scratch_types=[
            pltpu.VMEM((nb, EB), jnp.int32),
            pltpu.VMEM((nb, EB), jnp.int32),
            [pltpu.VMEM((EB, d), jnp.float32) for _ in range(NBUF)],
            pltpu.VMEM_SHARED((n_pad, d), jnp.float32),
            pltpu.VMEM_SHARED((n_tab, d), jnp.float32),
            [pltpu.SemaphoreType.DMA for _ in range(NBUF)],
            [pltpu.SemaphoreType.DMA for _ in range(NBUF)],
            pltpu.SemaphoreType.DMA,
        ],
    )
    def k(src_hbm, dst_hbm, table_hbm, zeros_hbm, out_hbm,
          sidx, didx, rows, acc, table, gsems, ssems, zsem):
        c = lax.axis_index("c")
        s = lax.axis_index("s")
        wid = c * NS + s
        r0 = s * rps
        tps = n_tab // NS
        zd = pltpu.async_copy(zeros_hbm.at[pl.ds(r0, rps)],
                              acc.at[pl.ds(r0, rps)], zsem)
        td = pltpu.async_copy(table_hbm.at[pl.ds(s * tps, tps)],
                              table.at[pl.ds(s * tps, tps)], zsem)
        pltpu.async_copy(src_hbm.at[wid], sidx, gsems[0]).wait()
        pltpu.async_copy(dst_hbm.at[wid], didx, gsems[1]).wait()
        zd.wait()
        td.wait()
        plsc.subcore_barrier()

        def gather(j, u):
            pltpu.async_copy(table.at[sidx.at[j]], rows[u], gsems[u])

        def wait_gather(u):
            pltpu.make_async_copy(table.at[sidx.at[0]], rows[u],
                                  gsems[u]).wait()

        def scatter(j, u):
            pltpu.async_copy(rows[u], acc.at[didx.at[j]], ssems[u], add=True)

        def wait_scatter(u):
            pltpu.make_async_copy(rows[u], acc.at[didx.at[0]], ssems[u]).wait()

        for u in range(NBUF):
            gather(u, u)

        def body(p, carry):
            j = NBUF * p
            for u in range(NBUF):
                wait_gather(u)
                scatter(j + u, u)
            nxt = j + NBUF

            @pl.when(nxt < nb)
            def _():
                for u in range(NBUF):
                    wait_scatter(u)
                    gather(nxt + u, u)

            return carry

        lax.fori_loop(0, nb // NBUF, body, 0)
        for u in range(NBUF):
            wait_scatter(u)
        plsc.subcore_barrier()
        pltpu.async_copy(acc.at[pl.ds(r0, rps)], out_hbm.at[c, pl.ds(r0, rps)],
                         zsem).wait()

    return k


def _tc_first(x, w0, d0, d1):
    n, din = x.shape
    dh = w0.shape[1]
    grid = (n // RB,)

    def body(x_ref, w_ref, d0_ref, d1_ref, g_ref, dis_ref):
        deg = 1.0 + d0_ref[:, 0:1] + d1_ref[:, 0:1]
        dis = lax.rsqrt(deg)
        h = _dot(x_ref[...], w_ref[...])
        g_ref[...] = h * dis
        dis_ref[...] = jnp.broadcast_to(dis, (RB, dh))

    return pl.pallas_call(
        body,
        grid=grid,
        in_specs=[
            pl.BlockSpec((RB, din), lambda i: (i, 0)),
            pl.BlockSpec((din, dh), lambda i: (0, 0)),
            pl.BlockSpec((RB, 8), lambda i: (i, 0)),
            pl.BlockSpec((RB, 8), lambda i: (i, 0)),
        ],
        out_specs=[
            pl.BlockSpec((RB, dh), lambda i: (i, 0)),
            pl.BlockSpec((RB, dh), lambda i: (i, 0)),
        ],
        out_shape=[
            jax.ShapeDtypeStruct((n, dh), jnp.float32),
            jax.ShapeDtypeStruct((n, dh), jnp.float32),
        ],
    )(x, w0, d0, d1)


def _tc_inter(a0, a1, g_prev, dis64, beta, lnw, lnb, w):
    n, dh = g_prev.shape

    def body(a0_ref, a1_ref, g_ref, dis_ref, beta_ref, lnw_ref, lnb_ref, w_ref,
             r_ref, gn_ref):
        dis = dis_ref[...]
        r = dis * (a0_ref[...] + a1_ref[...] + g_ref[...]) + beta_ref[...]
        r_ref[...] = r
        mu = jnp.mean(r, axis=-1, keepdims=True)
        cen = r - mu
        var = jnp.mean(cen * cen, axis=-1, keepdims=True)
        hn = cen * lax.rsqrt(var + 1e-5) * lnw_ref[...] + lnb_ref[...]
        h = jnp.where(hn >= 0, hn, 0.01 * hn)
        gn_ref[...] = dis * _dot(h, w_ref[...])

    return pl.pallas_call(
        body,
        grid=(n // RB,),
        in_specs=[
            pl.BlockSpec((RB, dh), lambda i: (i, 0)),
            pl.BlockSpec((RB, dh), lambda i: (i, 0)),
            pl.BlockSpec((RB, dh), lambda i: (i, 0)),
            pl.BlockSpec((RB, dh), lambda i: (i, 0)),
            pl.BlockSpec((1, dh), lambda i: (0, 0)),
            pl.BlockSpec((1, dh), lambda i: (0, 0)),
            pl.BlockSpec((1, dh), lambda i: (0, 0)),
            pl.BlockSpec((dh, dh), lambda i: (0, 0)),
        ],
        out_specs=[
            pl.BlockSpec((RB, dh), lambda i: (i, 0)),
            pl.BlockSpec((RB, dh), lambda i: (i, 0)),
        ],
        out_shape=[
            jax.ShapeDtypeStruct((n, dh), jnp.float32),
            jax.ShapeDtypeStruct((n, dh), jnp.float32),
        ],
    )(a0, a1, g_prev, dis64, beta, lnw, lnb, w)


def _tc_final(a0, a1, g_prev, dis64, beta):
    n, dh = g_prev.shape

    def body(a0_ref, a1_ref, g_ref, dis_ref, beta_ref, r_ref):
        r_ref[...] = dis_ref[...] * (a0_ref[...] + a1_ref[...] + g_ref[...]) \
            + beta_ref[...]

    return pl.pallas_call(
        body,
        grid=(n // RB,),
        in_specs=[
            pl.BlockSpec((RB, dh), lambda i: (i, 0)),
            pl.BlockSpec((RB, dh), lambda i: (i, 0)),
            pl.BlockSpec((RB, dh), lambda i: (i, 0)),
            pl.BlockSpec((RB, dh), lambda i: (i, 0)),
            pl.BlockSpec((1, dh), lambda i: (0, 0)),
        ],
        out_specs=[pl.BlockSpec((RB, dh), lambda i: (i, 0))],
        out_shape=[jax.ShapeDtypeStruct((n, dh), jnp.float32)],
    )(a0, a1, g_prev, dis64, beta)[0]


def _softmax_stats(x, batch_col, t, g):
    n, dtot = x.shape

    def body(x_ref, b_ref, t_ref, s1_ref, cnt_ref):
        i = pl.program_id(0)
        oh = (b_ref[...] == lax.broadcasted_iota(jnp.int32, (1, g), 1))
        oh = oh.astype(jnp.float32)
        s = t_ref[0, 0] * x_ref[...]
        p = _dot(oh, s, dims=((0,), (0,)))
        c = jnp.sum(oh, axis=0, keepdims=True)

        @pl.when(i == 0)
        def _():
            s1_ref[...] = p
            cnt_ref[...] = c

        @pl.when(i > 0)
        def _():
            s1_ref[...] += p
            cnt_ref[...] += c

    return pl.pallas_call(
        body,
        grid=(n // RB,),
        in_specs=[
            pl.BlockSpec((RB, dtot), lambda i: (i, 0)),
            pl.BlockSpec((RB, 1), lambda i: (i, 0)),
            pl.BlockSpec((1, 1), lambda i: (0, 0)),
        ],
        out_specs=[
            pl.BlockSpec((g, dtot), lambda i: (0, 0)),
            pl.BlockSpec((1, g), lambda i: (0, 0)),
        ],
        out_shape=[
            jax.ShapeDtypeStruct((g, dtot), jnp.float32),
            jax.ShapeDtypeStruct((1, g), jnp.float32),
        ],
    )(x, batch_col, t)


def _softmax_final(x, batch_col, t, s1, cnt_t, g):
    n, dtot = x.shape
    nblk = n // RB

    def body(x_ref, b_ref, t_ref, s1_ref, cnt_ref, out_ref, num_s, den_s):
        i = pl.program_id(0)
        oh = (b_ref[...] == lax.broadcasted_iota(jnp.int32, (1, g), 1))
        oh = oh.astype(jnp.float32)
        shift = s1_ref[...] / jnp.maximum(cnt_ref[...], 1.0)
        p = _dot(oh, shift)
        xv = x_ref[...]
        e = jnp.exp(t_ref[0, 0] * xv - p)
        num = _dot(oh, e * xv, dims=((0,), (0,)))
        den = _dot(oh, e, dims=((0,), (0,)))

        @pl.when(i == 0)
        def _():
            num_s[...] = num
            den_s[...] = den

        @pl.when(i > 0)
        def _():
            num_s[...] += num
            den_s[...] += den

        @pl.when(i == nblk - 1)
        def _():
            d = den_s[...]
            out_ref[...] = jnp.where(d > 0, num_s[...] / d, 0.0)

    return pl.pallas_call(
        body,
        grid=(nblk,),
        in_specs=[
            pl.BlockSpec((RB, dtot), lambda i: (i, 0)),
            pl.BlockSpec((RB, 1), lambda i: (i, 0)),
            pl.BlockSpec((1, 1), lambda i: (0, 0)),
            pl.BlockSpec((g, dtot), lambda i: (0, 0)),
            pl.BlockSpec((g, 1), lambda i: (0, 0)),
        ],
        out_specs=[pl.BlockSpec((g, dtot), lambda i: (0, 0))],
        out_shape=[jax.ShapeDtypeStruct((g, dtot), jnp.float32)],
        scratch_shapes=[
            pltpu.VMEM((g, dtot), jnp.float32),
            pltpu.VMEM((g, dtot), jnp.float32),
        ],
    )(x, batch_col, t, s1, cnt_t)[0]


def kernel(node_features, edge_index, edge_type_or_attr, batch_index,
           W0, b0, ln_w, ln_b, Ws, bs, t):
    n, din = node_features.shape
    e = edge_index.shape[1]
    dh = W0.shape[1]
    nlayers = Ws.shape[0]
    g = 64

    n_pad = ((n + NS * 8) // (NS * 8)) * (NS * 8)
    epw = ((e + NW - 1) // NW + NBUF * EB - 1) // (NBUF * EB) * (NBUF * EB)
    e_pad = epw * NW
    nb = epw // EB

    src = edge_index[0]
    dst = edge_index[1]
    pad = e_pad - e
    src_p = jnp.concatenate([src, jnp.zeros((pad,), jnp.int32)])
    dst_p = jnp.concatenate([dst, jnp.full((pad,), n_pad - 1, jnp.int32)])
    src_p = src_p.reshape(NW, nb, EB)
    dst_p = dst_p.reshape(NW, nb, EB)

    zeros16 = jnp.zeros((n_pad, 16), jnp.float32)
    zeros64 = jnp.zeros((n_pad, dh), jnp.float32)
    ones_blk = jnp.ones((EB, 16), jnp.float32)

    degk = _make_deg_kernel(n_pad, nb)
    scat64 = _make_scatter_kernel(n_pad, nb, dh, n)

    deg_parts = degk(dst_p, ones_blk, zeros16)
    d0 = deg_parts[0, :n, 0:8]
    d1 = deg_parts[1, :n, 0:8]

    g_cur, dis64 = _tc_first(node_features, W0, d0, d1)

    betas = [b0.reshape(1, dh)] + [bs[i].reshape(1, dh) for i in range(nlayers)]
    results = []
    for k in range(nlayers + 1):
        acc = scat64(src_p, dst_p, g_cur, zeros64)
        a0 = acc[0, :n, :]
        a1 = acc[1, :n, :]
        if k < nlayers:
            r, g_next = _tc_inter(a0, a1, g_cur, dis64, betas[k],
                                  ln_w[k].reshape(1, dh), ln_b[k].reshape(1, dh),
                                  Ws[k])
            results.append(r)
            g_cur = g_next
        else:
            results.append(_tc_final(a0, a1, g_cur, dis64, betas[k]))

    node_repr = jnp.concatenate(results, axis=-1)

    batch_col = batch_index.reshape(n, 1)
    t2 = t.reshape(1, 1)
    s1, cnt = _softmax_stats(node_repr, batch_col, t2, g)
    graph_repr = _softmax_final(node_repr, batch_col, t2, s1,
                                cnt.reshape(g, 1), g)
    return (graph_repr, node_repr)


# speedup vs baseline: 20.1230x; 1.0479x over previous
import functools

import jax
import jax.numpy as jnp
from jax import lax
from jax.experimental import pallas as pl
from jax.experimental.pallas import tpu as pltpu
from jax.experimental.pallas import tpu_sc as plsc

NC, NS = 2, 16
NW = NC * NS
EB = 128
NBUF = 2
RB = 2000

_HI = lax.Precision.HIGHEST


def _dot(a, b, dims=None):
    if dims is None:
        return jnp.dot(a, b, preferred_element_type=jnp.float32, precision=_HI)
    return lax.dot_general(a, b, (dims, ((), ())),
                           preferred_element_type=jnp.float32, precision=_HI)


@functools.lru_cache(maxsize=None)
def _make_deg_kernel(n_pad, nb):
    rps = n_pad // NS
    mesh = plsc.VectorSubcoreMesh(core_axis_name="c", subcore_axis_name="s",
                                  num_cores=NC, num_subcores=NS)

    @functools.partial(
        pl.kernel,
        mesh=mesh,
        compiler_params=pltpu.CompilerParams(use_tc_tiling_on_sc=False),
        out_type=jax.ShapeDtypeStruct((NC, n_pad, 16), jnp.float32),
        scratch_types=[
            pltpu.VMEM((nb, EB), jnp.int32),
            pltpu.VMEM((EB, 16), jnp.float32),
            pltpu.VMEM_SHARED((n_pad, 16), jnp.float32),
            [pltpu.SemaphoreType.DMA for _ in range(NBUF)],
            pltpu.SemaphoreType.DMA,
        ],
    )
    def k(dst_hbm, ones_hbm, zeros_hbm, out_hbm, didx, ones_v, acc, ssems,
          zsem):
        c = lax.axis_index("c")
        s = lax.axis_index("s")
        wid = c * NS + s
        r0 = s * rps
        zd = pltpu.async_copy(zeros_hbm.at[pl.ds(r0, rps)],
                              acc.at[pl.ds(r0, rps)], zsem)
        pltpu.async_copy(ones_hbm, ones_v, ssems[0]).wait()
        pltpu.async_copy(dst_hbm.at[wid], didx, ssems[0]).wait()
        zd.wait()
        plsc.subcore_barrier()

        def wait_scatter(u):
            pltpu.make_async_copy(ones_v, acc.at[didx.at[0]], ssems[u]).wait()

        def body(p, carry):
            j = NBUF * p
            for u in range(NBUF):
                pltpu.async_copy(ones_v, acc.at[didx.at[j + u]], ssems[u],
                                 add=True)
            for u in range(NBUF):
                wait_scatter(u)
            return carry

        lax.fori_loop(0, nb // NBUF, body, 0)
        plsc.subcore_barrier()
        pltpu.async_copy(acc.at[pl.ds(r0, rps)], out_hbm.at[c, pl.ds(r0, rps)],
                         zsem).wait()

    return k


@functools.lru_cache(maxsize=None)
def _make_scatter_kernel(n_pad, nb, d, n_tab):
    rps = n_pad // NS

    mesh = plsc.VectorSubcoreMesh(core_axis_name="c", subcore_axis_name="s",
                                  num_cores=NC, num_subcores=NS)

    @functools.partial(
        pl.kernel,
        mesh=mesh,
        compiler_params=pltpu.CompilerParams(use_tc_tiling_on_sc=False),
        out_type=jax.ShapeDtypeStruct((NC, n_pad, d), jnp.float32),
        scratch_types=[
            pltpu.VMEM((nb, EB), jnp.int32),
            pltpu.VMEM((nb, EB), jnp.int32),
            [pltpu.VMEM((EB, d), jnp.float32) for _ in range(NBUF)],
            pltpu.VMEM_SHARED((n_pad, d), jnp.float32),
            pltpu.VMEM_SHARED((n_tab, d), jnp.float32),
            [pltpu.SemaphoreType.DMA for _ in range(NBUF)],
            [pltpu.SemaphoreType.DMA for _ in range(NBUF)],
            pltpu.SemaphoreType.DMA,
        ],
    )
    def k(src_hbm, dst_hbm, table_hbm, zeros_hbm, out_hbm,
          sidx, didx, rows, acc, table, gsems, ssems, zsem):
        c = lax.axis_index("c")
        s = lax.axis_index("s")
        wid = c * NS + s
        r0 = s * rps
        tps = n_tab // NS
        zd = pltpu.async_copy(zeros_hbm.at[pl.ds(r0, rps)],
                              acc.at[pl.ds(r0, rps)], zsem)
        td = pltpu.async_copy(table_hbm.at[pl.ds(s * tps, tps)],
                              table.at[pl.ds(s * tps, tps)], zsem)
        pltpu.async_copy(src_hbm.at[wid], sidx, gsems[0]).wait()
        pltpu.async_copy(dst_hbm.at[wid], didx, gsems[1]).wait()
        zd.wait()
        td.wait()
        plsc.subcore_barrier()

        def gather(j, u):
            pltpu.async_copy(table.at[sidx.at[j]], rows[u], gsems[u])

        def wait_gather(u):
            pltpu.make_async_copy(table.at[sidx.at[0]], rows[u],
                                  gsems[u]).wait()

        def scatter(j, u):
            pltpu.async_copy(rows[u], acc.at[didx.at[j]], ssems[u], add=True)

        def wait_scatter(u):
            pltpu.make_async_copy(rows[u], acc.at[didx.at[0]], ssems[u]).wait()

        for u in range(NBUF):
            gather(u, u)

        def body(p, carry):
            j = NBUF * p
            for u in range(NBUF):
                wait_gather(u)
                scatter(j + u, u)
            nxt = j + NBUF

            @pl.when(nxt < nb)
            def _():
                for u in range(NBUF):
                    wait_scatter(u)
                    gather(nxt + u, u)

            return carry

        lax.fori_loop(0, nb // NBUF, body, 0)
        for u in range(NBUF):
            wait_scatter(u)
        plsc.subcore_barrier()
        pltpu.async_copy(acc.at[pl.ds(r0, rps)], out_hbm.at[c, pl.ds(r0, rps)],
                         zsem).wait()

    return k


def _tc_first(x, w0, d0, d1):
    n, din = x.shape
    dh = w0.shape[1]
    grid = (n // RB,)

    def body(x_ref, w_ref, d0_ref, d1_ref, g_ref, dis_ref):
        deg = 1.0 + d0_ref[:, 0:1] + d1_ref[:, 0:1]
        dis = lax.rsqrt(deg)
        h = _dot(x_ref[...], w_ref[...])
        g_ref[...] = h * dis
        dis_ref[...] = jnp.broadcast_to(dis, (RB, dh))

    return pl.pallas_call(
        body,
        grid=grid,
        in_specs=[
            pl.BlockSpec((RB, din), lambda i: (i, 0)),
            pl.BlockSpec((din, dh), lambda i: (0, 0)),
            pl.BlockSpec((RB, 8), lambda i: (i, 0)),
            pl.BlockSpec((RB, 8), lambda i: (i, 0)),
        ],
        out_specs=[
            pl.BlockSpec((RB, dh), lambda i: (i, 0)),
            pl.BlockSpec((RB, dh), lambda i: (i, 0)),
        ],
        out_shape=[
            jax.ShapeDtypeStruct((n, dh), jnp.float32),
            jax.ShapeDtypeStruct((n, dh), jnp.float32),
        ],
    )(x, w0, d0, d1)


def _tc_inter(a0, a1, g_prev, dis64, beta, lnw, lnb, w):
    n, dh = g_prev.shape

    def body(a0_ref, a1_ref, g_ref, dis_ref, beta_ref, lnw_ref, lnb_ref, w_ref,
             r_ref, gn_ref):
        dis = dis_ref[...]
        r = dis * (a0_ref[...] + a1_ref[...] + g_ref[...]) + beta_ref[...]
        r_ref[...] = r
        mu = jnp.mean(r, axis=-1, keepdims=True)
        cen = r - mu
        var = jnp.mean(cen * cen, axis=-1, keepdims=True)
        hn = cen * lax.rsqrt(var + 1e-5) * lnw_ref[...] + lnb_ref[...]
        h = jnp.where(hn >= 0, hn, 0.01 * hn)
        gn_ref[...] = dis * _dot(h, w_ref[...])

    return pl.pallas_call(
        body,
        grid=(n // RB,),
        in_specs=[
            pl.BlockSpec((RB, dh), lambda i: (i, 0)),
            pl.BlockSpec((RB, dh), lambda i: (i, 0)),
            pl.BlockSpec((RB, dh), lambda i: (i, 0)),
            pl.BlockSpec((RB, dh), lambda i: (i, 0)),
            pl.BlockSpec((1, dh), lambda i: (0, 0)),
            pl.BlockSpec((1, dh), lambda i: (0, 0)),
            pl.BlockSpec((1, dh), lambda i: (0, 0)),
            pl.BlockSpec((dh, dh), lambda i: (0, 0)),
        ],
        out_specs=[
            pl.BlockSpec((RB, dh), lambda i: (i, 0)),
            pl.BlockSpec((RB, dh), lambda i: (i, 0)),
        ],
        out_shape=[
            jax.ShapeDtypeStruct((n, dh), jnp.float32),
            jax.ShapeDtypeStruct((n, dh), jnp.float32),
        ],
    )(a0, a1, g_prev, dis64, beta, lnw, lnb, w)


def _softmax_stats(rs, a0, a1, g_last, dis64, beta, batch_col, t, g):
    n, dh = g_last.shape
    nr = len(rs)
    dtot = (nr + 1) * dh

    def body(*refs):
        rrefs = refs[:nr]
        (a0_ref, a1_ref, gl_ref, dis_ref, beta_ref, b_ref, t_ref,
         x_ref, s1_ref, cnt_ref) = refs[nr:]
        i = pl.program_id(0)
        r_last = dis_ref[...] * (a0_ref[...] + a1_ref[...] + gl_ref[...]) \
            + beta_ref[...]
        x = jnp.concatenate([r[...] for r in rrefs] + [r_last], axis=-1)
        x_ref[...] = x
        oh = (b_ref[...] == lax.broadcasted_iota(jnp.int32, (1, g), 1))
        oh = oh.astype(jnp.float32)
        s = t_ref[0, 0] * x
        p = _dot(oh, s, dims=((0,), (0,)))
        c = jnp.sum(oh, axis=0, keepdims=True)

        @pl.when(i == 0)
        def _():
            s1_ref[...] = p
            cnt_ref[...] = c

        @pl.when(i > 0)
        def _():
            s1_ref[...] += p
            cnt_ref[...] += c

    blk64 = pl.BlockSpec((RB, dh), lambda i: (i, 0))
    vec = pl.BlockSpec((1, dh), lambda i: (0, 0))
    return pl.pallas_call(
        body,
        grid=(n // RB,),
        in_specs=[blk64] * nr + [
            blk64, blk64, blk64, blk64, vec,
            pl.BlockSpec((RB, 1), lambda i: (i, 0)),
            pl.BlockSpec((1, 1), lambda i: (0, 0)),
        ],
        out_specs=[
            pl.BlockSpec((RB, dtot), lambda i: (i, 0)),
            pl.BlockSpec((g, dtot), lambda i: (0, 0)),
            pl.BlockSpec((1, g), lambda i: (0, 0)),
        ],
        out_shape=[
            jax.ShapeDtypeStruct((n, dtot), jnp.float32),
            jax.ShapeDtypeStruct((g, dtot), jnp.float32),
            jax.ShapeDtypeStruct((1, g), jnp.float32),
        ],
    )(*rs, a0, a1, g_last, dis64, beta, batch_col, t)


def _softmax_final(x, batch_col, t, s1, cnt_t, g):
    n, dtot = x.shape
    nblk = n // RB

    def body(x_ref, b_ref, t_ref, s1_ref, cnt_ref, out_ref, num_s, den_s):
        i = pl.program_id(0)
        oh = (b_ref[...] == lax.broadcasted_iota(jnp.int32, (1, g), 1))
        oh = oh.astype(jnp.float32)
        shift = s1_ref[...] / jnp.maximum(cnt_ref[...], 1.0)
        p = _dot(oh, shift)
        xv = x_ref[...]
        e = jnp.exp(t_ref[0, 0] * xv - p)
        num = _dot(oh, e * xv, dims=((0,), (0,)))
        den = _dot(oh, e, dims=((0,), (0,)))

        @pl.when(i == 0)
        def _():
            num_s[...] = num
            den_s[...] = den

        @pl.when(i > 0)
        def _():
            num_s[...] += num
            den_s[...] += den

        @pl.when(i == nblk - 1)
        def _():
            d = den_s[...]
            out_ref[...] = jnp.where(d > 0, num_s[...] / d, 0.0)

    return pl.pallas_call(
        body,
        grid=(nblk,),
        in_specs=[
            pl.BlockSpec((RB, dtot), lambda i: (i, 0)),
            pl.BlockSpec((RB, 1), lambda i: (i, 0)),
            pl.BlockSpec((1, 1), lambda i: (0, 0)),
            pl.BlockSpec((g, dtot), lambda i: (0, 0)),
            pl.BlockSpec((g, 1), lambda i: (0, 0)),
        ],
        out_specs=[pl.BlockSpec((g, dtot), lambda i: (0, 0))],
        out_shape=[jax.ShapeDtypeStruct((g, dtot), jnp.float32)],
        scratch_shapes=[
            pltpu.VMEM((g, dtot), jnp.float32),
            pltpu.VMEM((g, dtot), jnp.float32),
        ],
    )(x, batch_col, t, s1, cnt_t)[0]


def kernel(node_features, edge_index, edge_type_or_attr, batch_index,
           W0, b0, ln_w, ln_b, Ws, bs, t):
    n, din = node_features.shape
    e = edge_index.shape[1]
    dh = W0.shape[1]
    nlayers = Ws.shape[0]
    g = 64

    n_pad = ((n + NS * 8) // (NS * 8)) * (NS * 8)
    epw = ((e + NW - 1) // NW + NBUF * EB - 1) // (NBUF * EB) * (NBUF * EB)
    e_pad = epw * NW
    nb = epw // EB

    src = edge_index[0]
    dst = edge_index[1]
    pad = e_pad - e
    src_p = jnp.concatenate([src, jnp.zeros((pad,), jnp.int32)])
    dst_p = jnp.concatenate([dst, jnp.full((pad,), n_pad - 1, jnp.int32)])
    src_p = src_p.reshape(NW, nb, EB)
    dst_p = dst_p.reshape(NW, nb, EB)

    zeros16 = jnp.zeros((n_pad, 16), jnp.float32)
    zeros64 = jnp.zeros((n_pad, dh), jnp.float32)
    ones_blk = jnp.ones((EB, 16), jnp.float32)

    degk = _make_deg_kernel(n_pad, nb)
    scat64 = _make_scatter_kernel(n_pad, nb, dh, n)

    deg_parts = degk(dst_p, ones_blk, zeros16)
    d0 = deg_parts[0, :n, 0:8]
    d1 = deg_parts[1, :n, 0:8]

    g_cur, dis64 = _tc_first(node_features, W0, d0, d1)

    betas = [b0.reshape(1, dh)] + [bs[i].reshape(1, dh) for i in range(nlayers)]
    results = []
    for k in range(nlayers + 1):
        acc = scat64(src_p, dst_p, g_cur, zeros64)
        a0 = acc[0, :n, :]
        a1 = acc[1, :n, :]
        if k < nlayers:
            r, g_next = _tc_inter(a0, a1, g_cur, dis64, betas[k],
                                  ln_w[k].reshape(1, dh), ln_b[k].reshape(1, dh),
                                  Ws[k])
            results.append(r)
            g_cur = g_next

    batch_col = batch_index.reshape(n, 1)
    t2 = t.reshape(1, 1)
    node_repr, s1, cnt = _softmax_stats(results, a0, a1, g_cur, dis64,
                                        betas[nlayers], batch_col, t2, g)
    graph_repr = _softmax_final(node_repr, batch_col, t2, s1,
                                cnt.reshape(g, 1), g)
    return (graph_repr, node_repr)


# speedup vs baseline: 21.1386x; 1.0505x over previous
import functools

import jax
import jax.numpy as jnp
from jax import lax
from jax.experimental import pallas as pl
from jax.experimental.pallas import tpu as pltpu
from jax.experimental.pallas import tpu_sc as plsc

NC, NS = 2, 16
NW = NC * NS
EB = 128
NBUF = 2
RB = 2000

_HI = lax.Precision.HIGHEST


def _dot(a, b, dims=None):
    if dims is None:
        return jnp.dot(a, b, preferred_element_type=jnp.float32, precision=_HI)
    return lax.dot_general(a, b, (dims, ((), ())),
                           preferred_element_type=jnp.float32, precision=_HI)


@functools.lru_cache(maxsize=None)
def _make_deg_kernel(n_pad, nb):
    rps = n_pad // NS
    mesh = plsc.VectorSubcoreMesh(core_axis_name="c", subcore_axis_name="s",
                                  num_cores=NC, num_subcores=NS)

    @functools.partial(
        pl.kernel,
        mesh=mesh,
        compiler_params=pltpu.CompilerParams(use_tc_tiling_on_sc=False),
        out_type=jax.ShapeDtypeStruct((NC, n_pad, 16), jnp.float32),
        scratch_types=[
            pltpu.VMEM((nb, EB), jnp.int32),
            pltpu.VMEM((EB, 16), jnp.float32),
            pltpu.VMEM_SHARED((n_pad, 16), jnp.float32),
            [pltpu.SemaphoreType.DMA for _ in range(NBUF)],
            pltpu.SemaphoreType.DMA,
        ],
    )
    def k(dst_hbm, ones_hbm, zeros_hbm, out_hbm, didx, ones_v, acc, ssems,
          zsem):
        c = lax.axis_index("c")
        s = lax.axis_index("s")
        wid = c * NS + s
        r0 = s * rps
        zd = pltpu.async_copy(zeros_hbm.at[pl.ds(r0, rps)],
                              acc.at[pl.ds(r0, rps)], zsem)
        pltpu.async_copy(ones_hbm, ones_v, ssems[0]).wait()
        pltpu.async_copy(dst_hbm.at[wid], didx, ssems[0]).wait()
        zd.wait()
        plsc.subcore_barrier()

        def wait_scatter(u):
            pltpu.make_async_copy(ones_v, acc.at[didx.at[0]], ssems[u]).wait()

        def body(p, carry):
            j = NBUF * p
            for u in range(NBUF):
                pltpu.async_copy(ones_v, acc.at[didx.at[j + u]], ssems[u],
                                 add=True)
            for u in range(NBUF):
                wait_scatter(u)
            return carry

        lax.fori_loop(0, nb // NBUF, body, 0)
        plsc.subcore_barrier()
        pltpu.async_copy(acc.at[pl.ds(r0, rps)], out_hbm.at[c, pl.ds(r0, rps)],
                         zsem).wait()

    return k


@functools.lru_cache(maxsize=None)
def _make_scatter_kernel(n_pad, nb, d, n_tab):
    rps = n_pad // NS

    mesh = plsc.VectorSubcoreMesh(core_axis_name="c", subcore_axis_name="s",
                                  num_cores=NC, num_subcores=NS)

    @functools.partial(
        pl.kernel,
        mesh=mesh,
        compiler_params=pltpu.CompilerParams(use_tc_tiling_on_sc=False),
        out_type=jax.ShapeDtypeStruct((NC, n_pad, d), jnp.float32),
        scratch_types=[
            pltpu.VMEM((nb, EB), jnp.int32),
            pltpu.VMEM((nb, EB), jnp.int32),
            [pltpu.VMEM((EB, d), jnp.float32) for _ in range(NBUF)],
            pltpu.VMEM_SHARED((n_pad, d), jnp.float32),
            pltpu.VMEM_SHARED((n_tab, d), jnp.float32),
            [pltpu.SemaphoreType.DMA for _ in range(NBUF)],
            [pltpu.SemaphoreType.DMA for _ in range(NBUF)],
            pltpu.SemaphoreType.DMA,
        ],
    )
    def k(src_hbm, dst_hbm, table_hbm, zeros_hbm, out_hbm,
          sidx, didx, rows, acc, table, gsems, ssems, zsem):
        c = lax.axis_index("c")
        s = lax.axis_index("s")
        wid = c * NS + s
        r0 = s * rps
        tps = n_tab // NS
        zd = pltpu.async_copy(zeros_hbm.at[pl.ds(r0, rps)],
                              acc.at[pl.ds(r0, rps)], zsem)
        td = pltpu.async_copy(table_hbm.at[pl.ds(s * tps, tps)],
                              table.at[pl.ds(s * tps, tps)], zsem)
        pltpu.async_copy(src_hbm.at[wid], sidx, gsems[0]).wait()
        pltpu.async_copy(dst_hbm.at[wid], didx, gsems[1]).wait()
        zd.wait()
        td.wait()
        plsc.subcore_barrier()

        def gather(j, u):
            pltpu.async_copy(table.at[sidx.at[j]], rows[u], gsems[u])

        def wait_gather(u):
            pltpu.make_async_copy(table.at[sidx.at[0]], rows[u],
                                  gsems[u]).wait()

        def scatter(j, u):
            pltpu.async_copy(rows[u], acc.at[didx.at[j]], ssems[u], add=True)

        def wait_scatter(u):
            pltpu.make_async_copy(rows[u], acc.at[didx.at[0]], ssems[u]).wait()

        for u in range(NBUF):
            gather(u, u)

        def body(p, carry):
            j = NBUF * p
            for u in range(NBUF):
                wait_gather(u)
                scatter(j + u, u)
            nxt = j + NBUF

            @pl.when(nxt < nb)
            def _():
                for u in range(NBUF):
                    wait_scatter(u)
                    gather(nxt + u, u)

            return carry

        lax.fori_loop(0, nb // NBUF, body, 0)
        for u in range(NBUF):
            wait_scatter(u)
        plsc.subcore_barrier()
        pltpu.async_copy(acc.at[pl.ds(r0, rps)], out_hbm.at[c, pl.ds(r0, rps)],
                         zsem).wait()

    return k


def _tc_first(x, w0, deg_parts):
    n, din = x.shape
    dh = w0.shape[1]
    grid = (n // RB,)

    def body(x_ref, w_ref, dp_ref, g_ref, dis_ref):
        deg = 1.0 + dp_ref[0, :, 0:1] + dp_ref[1, :, 0:1]
        dis = lax.rsqrt(deg)
        h = _dot(x_ref[...], w_ref[...])
        g_ref[...] = h * dis
        dis_ref[...] = jnp.broadcast_to(dis, (RB, dh))

    return pl.pallas_call(
        body,
        grid=grid,
        in_specs=[
            pl.BlockSpec((RB, din), lambda i: (i, 0)),
            pl.BlockSpec((din, dh), lambda i: (0, 0)),
            pl.BlockSpec((2, RB, 16), lambda i: (0, i, 0)),
        ],
        out_specs=[
            pl.BlockSpec((RB, dh), lambda i: (i, 0)),
            pl.BlockSpec((RB, dh), lambda i: (i, 0)),
        ],
        out_shape=[
            jax.ShapeDtypeStruct((n, dh), jnp.float32),
            jax.ShapeDtypeStruct((n, dh), jnp.float32),
        ],
    )(x, w0, deg_parts)


def _tc_inter(acc, g_prev, dis64, beta, lnw, lnb, w):
    n, dh = g_prev.shape

    def body(acc_ref, g_ref, dis_ref, beta_ref, lnw_ref, lnb_ref, w_ref,
             r_ref, gn_ref):
        dis = dis_ref[...]
        r = dis * (acc_ref[0] + acc_ref[1] + g_ref[...]) + beta_ref[...]
        r_ref[...] = r
        mu = jnp.mean(r, axis=-1, keepdims=True)
        cen = r - mu
        var = jnp.mean(cen * cen, axis=-1, keepdims=True)
        hn = cen * lax.rsqrt(var + 1e-5) * lnw_ref[...] + lnb_ref[...]
        h = jnp.where(hn >= 0, hn, 0.01 * hn)
        gn_ref[...] = dis * _dot(h, w_ref[...])

    return pl.pallas_call(
        body,
        grid=(n // RB,),
        in_specs=[
            pl.BlockSpec((2, RB, dh), lambda i: (0, i, 0)),
            pl.BlockSpec((RB, dh), lambda i: (i, 0)),
            pl.BlockSpec((RB, dh), lambda i: (i, 0)),
            pl.BlockSpec((1, dh), lambda i: (0, 0)),
            pl.BlockSpec((1, dh), lambda i: (0, 0)),
            pl.BlockSpec((1, dh), lambda i: (0, 0)),
            pl.BlockSpec((dh, dh), lambda i: (0, 0)),
        ],
        out_specs=[
            pl.BlockSpec((RB, dh), lambda i: (i, 0)),
            pl.BlockSpec((RB, dh), lambda i: (i, 0)),
        ],
        out_shape=[
            jax.ShapeDtypeStruct((n, dh), jnp.float32),
            jax.ShapeDtypeStruct((n, dh), jnp.float32),
        ],
    )(acc, g_prev, dis64, beta, lnw, lnb, w)


def _softmax_stats(rs, acc, g_last, dis64, beta, batch_col, t, g):
    n, dh = g_last.shape
    nr = len(rs)
    dtot = (nr + 1) * dh

    def body(*refs):
        rrefs = refs[:nr]
        (acc_ref, gl_ref, dis_ref, beta_ref, b_ref, t_ref,
         x_ref, s1_ref, cnt_ref) = refs[nr:]
        i = pl.program_id(0)
        r_last = dis_ref[...] * (acc_ref[0] + acc_ref[1] + gl_ref[...]) \
            + beta_ref[...]
        x = jnp.concatenate([r[...] for r in rrefs] + [r_last], axis=-1)
        x_ref[...] = x
        oh = (b_ref[...] == lax.broadcasted_iota(jnp.int32, (1, g), 1))
        oh = oh.astype(jnp.float32)
        s = t_ref[0, 0] * x
        p = _dot(oh, s, dims=((0,), (0,)))
        c = jnp.sum(oh, axis=0, keepdims=True)

        @pl.when(i == 0)
        def _():
            s1_ref[...] = p
            cnt_ref[...] = c

        @pl.when(i > 0)
        def _():
            s1_ref[...] += p
            cnt_ref[...] += c

    blk64 = pl.BlockSpec((RB, dh), lambda i: (i, 0))
    vec = pl.BlockSpec((1, dh), lambda i: (0, 0))
    return pl.pallas_call(
        body,
        grid=(n // RB,),
        in_specs=[blk64] * nr + [
            pl.BlockSpec((2, RB, dh), lambda i: (0, i, 0)),
            blk64, blk64, vec,
            pl.BlockSpec((RB, 1), lambda i: (i, 0)),
            pl.BlockSpec((1, 1), lambda i: (0, 0)),
        ],
        out_specs=[
            pl.BlockSpec((RB, dtot), lambda i: (i, 0)),
            pl.BlockSpec((g, dtot), lambda i: (0, 0)),
            pl.BlockSpec((1, g), lambda i: (0, 0)),
        ],
        out_shape=[
            jax.ShapeDtypeStruct((n, dtot), jnp.float32),
            jax.ShapeDtypeStruct((g, dtot), jnp.float32),
            jax.ShapeDtypeStruct((1, g), jnp.float32),
        ],
    )(*rs, acc, g_last, dis64, beta, batch_col, t)


def _softmax_final(x, batch_col, t, s1, cnt_t, g):
    n, dtot = x.shape
    nblk = n // RB

    def body(x_ref, b_ref, t_ref, s1_ref, cnt_ref, out_ref, num_s, den_s):
        i = pl.program_id(0)
        oh = (b_ref[...] == lax.broadcasted_iota(jnp.int32, (1, g), 1))
        oh = oh.astype(jnp.float32)
        shift = s1_ref[...] / jnp.maximum(cnt_ref[...], 1.0)
        p = _dot(oh, shift)
        xv = x_ref[...]
        e = jnp.exp(t_ref[0, 0] * xv - p)
        num = _dot(oh, e * xv, dims=((0,), (0,)))
        den = _dot(oh, e, dims=((0,), (0,)))

        @pl.when(i == 0)
        def _():
            num_s[...] = num
            den_s[...] = den

        @pl.when(i > 0)
        def _():
            num_s[...] += num
            den_s[...] += den

        @pl.when(i == nblk - 1)
        def _():
            d = den_s[...]
            out_ref[...] = jnp.where(d > 0, num_s[...] / d, 0.0)

    return pl.pallas_call(
        body,
        grid=(nblk,),
        in_specs=[
            pl.BlockSpec((RB, dtot), lambda i: (i, 0)),
            pl.BlockSpec((RB, 1), lambda i: (i, 0)),
            pl.BlockSpec((1, 1), lambda i: (0, 0)),
            pl.BlockSpec((g, dtot), lambda i: (0, 0)),
            pl.BlockSpec((g, 1), lambda i: (0, 0)),
        ],
        out_specs=[pl.BlockSpec((g, dtot), lambda i: (0, 0))],
        out_shape=[jax.ShapeDtypeStruct((g, dtot), jnp.float32)],
        scratch_shapes=[
            pltpu.VMEM((g, dtot), jnp.float32),
            pltpu.VMEM((g, dtot), jnp.float32),
        ],
    )(x, batch_col, t, s1, cnt_t)[0]


def kernel(node_features, edge_index, edge_type_or_attr, batch_index,
           W0, b0, ln_w, ln_b, Ws, bs, t):
    n, din = node_features.shape
    e = edge_index.shape[1]
    dh = W0.shape[1]
    nlayers = Ws.shape[0]
    g = 64

    n_pad = ((n + NS * 8) // (NS * 8)) * (NS * 8)
    epw = ((e + NW - 1) // NW + NBUF * EB - 1) // (NBUF * EB) * (NBUF * EB)
    e_pad = epw * NW
    nb = epw // EB

    src = edge_index[0]
    dst = edge_index[1]
    pad = e_pad - e
    src_p = jnp.concatenate([src, jnp.zeros((pad,), jnp.int32)])
    dst_p = jnp.concatenate([dst, jnp.full((pad,), n_pad - 1, jnp.int32)])
    src_p = src_p.reshape(NW, nb, EB)
    dst_p = dst_p.reshape(NW, nb, EB)

    zeros16 = jnp.zeros((n_pad, 16), jnp.float32)
    zeros64 = jnp.zeros((n_pad, dh), jnp.float32)
    ones_blk = jnp.ones((EB, 16), jnp.float32)

    degk = _make_deg_kernel(n_pad, nb)
    scat64 = _make_scatter_kernel(n_pad, nb, dh, n)

    deg_parts = degk(dst_p, ones_blk, zeros16)

    g_cur, dis64 = _tc_first(node_features, W0, deg_parts)

    betas = [b0.reshape(1, dh)] + [bs[i].reshape(1, dh) for i in range(nlayers)]
    results = []
    for k in range(nlayers + 1):
        acc = scat64(src_p, dst_p, g_cur, zeros64)
        if k < nlayers:
            r, g_next = _tc_inter(acc, g_cur, dis64, betas[k],
                                  ln_w[k].reshape(1, dh), ln_b[k].reshape(1, dh),
                                  Ws[k])
            results.append(r)
            g_cur = g_next

    batch_col = batch_index.reshape(n, 1)
    t2 = t.reshape(1, 1)
    node_repr, s1, cnt = _softmax_stats(results, acc, g_cur, dis64,
                                        betas[nlayers], batch_col, t2, g)
    graph_repr = _softmax_final(node_repr, batch_col, t2, s1,
                                cnt.reshape(g, 1), g)
    return (graph_repr, node_repr)
